# baseline (speedup 1.0000x reference)
import numpy as np
import ml_dtypes
import jax
import jax.numpy as jnp
from jax.experimental import pallas as pl

_EPS = float(np.finfo('float').eps)
_HW = 64
_R = 6


def _build_tables():
    e = (np.arange(_HW, dtype=np.float32) / np.float32(5.0)).astype(np.float32)
    b = e.astype(ml_dtypes.bfloat16).astype(np.float32)
    d2 = (e[:, None] * e[:, None] + e[None, :] * e[None, :]
          - np.float32(2.0) * (b[:, None] * b[None, :])).astype(np.float32)
    gb = np.exp(-0.5 * d2.astype(np.float64)).astype(np.float32)

    offsets = []
    corrs = []
    idx = np.arange(_HW)
    for dy in range(-_R, _R + 1):
        jy = idx + dy
        vy = (jy >= 0) & (jy < _HW)
        u = np.full(_HW, np.inf, np.float32)
        u[vy] = d2[idx[vy], jy[vy]]
        for dx in range(-_R, _R + 1):
            jx = idx + dx
            vx = (jx >= 0) & (jx < _HW)
            v = np.full(_HW, np.inf, np.float32)
            v[vx] = d2[idx[vx], jx[vx]]
            s = u[:, None] + v[None, :]
            neg = s < 0
            if neg.any():
                c = np.zeros((_HW, _HW), np.float64)
                c[neg] = 1.0 - np.exp(-0.5 * s[neg].astype(np.float64))
                offsets.append((dy, dx))
                corrs.append(c.astype(np.float32))
    order = sorted(range(len(offsets)), key=lambda i: (offsets[i][1], offsets[i][0]))
    offsets = [offsets[i] for i in order]
    corrs = [corrs[i] for i in order]
    corr = np.stack(corrs)

    sg = gb.astype(np.float64).sum(axis=1)
    norm = np.outer(sg, sg) + corr.astype(np.float64).sum(axis=0)
    inv_norm = (1.0 / (norm + _EPS)).astype(np.float32)
    return gb, offsets, corr, inv_norm


_GB_NP, _OFFSETS, _CORR_NP, _INVN_NP = _build_tables()


def _filter_kernel(q_ref, qsh_ref, gb_ref, corr_ref, invn_ref, o_ref):
    d = o_ref.shape[0]
    gb = gb_ref[...]
    q = q_ref[...]

    t1 = jnp.dot(q.reshape(d * _HW, _HW), gb,
                 preferred_element_type=jnp.float32).reshape(d, _HW, _HW)
    acc = jnp.stack([
        jnp.dot(gb, t1[c], preferred_element_type=jnp.float32)
        for c in range(d)
    ])

    for k, (dy, dx) in enumerate(_OFFSETS):
        qs = qsh_ref[dx + _R, :, _R + dy:_R + dy + _HW, :]
        acc = acc + corr_ref[k][None, :, :] * qs

    o_ref[...] = acc * invn_ref[...][None, :, :]


def kernel(input_, image):
    _, d, h, w = input_.shape
    q = input_.reshape(d, h, w)
    qpad = jnp.pad(q, ((0, 0), (_R, _R), (_R, _R)))
    qsh = jnp.stack([qpad[:, :, _R + dx:_R + dx + w]
                     for dx in range(-_R, _R + 1)])
    out = pl.pallas_call(
        _filter_kernel,
        out_shape=jax.ShapeDtypeStruct((d, h, w), jnp.float32),
    )(q, qsh, jnp.asarray(_GB_NP), jnp.asarray(_CORR_NP), jnp.asarray(_INVN_NP))
    return out.reshape(1, d, h, w)


# speedup vs baseline: 2.8657x; 2.8657x over previous
import numpy as np
import ml_dtypes
import jax
import jax.numpy as jnp
from jax.experimental import pallas as pl
from jax.experimental.pallas import tpu as pltpu

_EPS = float(np.finfo('float').eps)
_HW = 64
_R = 6
_PROWS = _HW // 2
_PPAD = _PROWS + 2 * (_R // 2)


def _build_tables():
    e = (np.arange(_HW, dtype=np.float32) / np.float32(5.0)).astype(np.float32)
    b = e.astype(ml_dtypes.bfloat16).astype(np.float32)
    d2 = (e[:, None] * e[:, None] + e[None, :] * e[None, :]
          - np.float32(2.0) * (b[:, None] * b[None, :])).astype(np.float32)
    gb = np.exp(-0.5 * d2.astype(np.float64)).astype(np.float32)

    offsets = []
    corrs = []
    idx = np.arange(_HW)
    for dy in range(-_R, _R + 1):
        jy = idx + dy
        vy = (jy >= 0) & (jy < _HW)
        u = np.full(_HW, np.inf, np.float32)
        u[vy] = d2[idx[vy], jy[vy]]
        for dx in range(-_R, _R + 1):
            jx = idx + dx
            vx = (jx >= 0) & (jx < _HW)
            v = np.full(_HW, np.inf, np.float32)
            v[vx] = d2[idx[vx], jx[vx]]
            s = u[:, None] + v[None, :]
            neg = s < 0
            if neg.any():
                c = np.zeros((_HW, _HW), np.float64)
                c[neg] = 1.0 - np.exp(-0.5 * s[neg].astype(np.float64))
                offsets.append((dy, dx))
                corrs.append(c.astype(np.float32))

    sg = gb.astype(np.float64).sum(axis=1)
    norm = np.outer(sg, sg) + np.stack(corrs).astype(np.float64).sum(axis=0)
    inv_norm = (1.0 / (norm + _EPS)).astype(np.float32).reshape(_PROWS, 2 * _HW)

    gb2 = np.zeros((2 * _HW, 2 * _HW), np.float32)
    gb2[:_HW, :_HW] = gb
    gb2[_HW:, _HW:] = gb
    geo = np.zeros((_HW, _HW), np.float32)
    geo[:_PROWS, :_PROWS] = gb[0::2, 0::2]
    geo[:_PROWS, _PROWS:] = gb[0::2, 1::2]
    geo[_PROWS:, :_PROWS] = gb[1::2, 1::2]
    geo[_PROWS:, _PROWS:] = gb[1::2, 0::2]

    taps = []
    corr_fields = []
    for (dy, dx), c in zip(offsets, corrs):
        cp = c.reshape(_PROWS, 2 * _HW)
        g = dx + _R
        if dy % 2 == 0:
            m = (dy + 2 * (_R // 2)) // 2
            taps.append((2 * g, m, len(corr_fields)))
            corr_fields.append(cp)
        else:
            m = (dy + 2 * (_R // 2) - 1) // 2
            c0 = cp.copy()
            c0[:, _HW:] = 0.0
            c1 = cp.copy()
            c1[:, :_HW] = 0.0
            taps.append((2 * g + 1, m, len(corr_fields)))
            corr_fields.append(c0)
            taps.append((2 * g + 1, m + 1, len(corr_fields)))
            corr_fields.append(c1)
    corr = np.stack(corr_fields).astype(ml_dtypes.bfloat16)
    return gb2, geo, taps, corr, inv_norm


_GB2_NP, _GEO_NP, _TAPS, _CORR_NP, _INVN_NP = _build_tables()


def _filter_kernel(qlive_ref, qpk_ref, gb2_ref, geo_ref, corr_ref, invn_ref,
                   o_ref, scr_ref):
    d = qlive_ref.shape[0]
    lanes = 2 * _HW
    qlive = qlive_ref[...]

    t1p = jnp.dot(qlive.reshape(d * _PROWS, lanes), gb2_ref[...],
                  preferred_element_type=jnp.float32).reshape(d, _PROWS, lanes)
    t1sw = pltpu.roll(t1p, _HW, 2)
    s2 = jnp.concatenate([t1p, t1sw], axis=1)
    lane_is_low = jax.lax.broadcasted_iota(
        jnp.int32, (_PROWS, lanes), 1) < _HW
    geo = geo_ref[...]
    sep = [
        jnp.where(lane_is_low,
                  *jnp.split(jnp.dot(geo, s2[c],
                                     preferred_element_type=jnp.float32), 2))
        for c in range(d)
    ]
    acc = jnp.stack(sep)

    qpk = qpk_ref[...]
    for g in range(2 * _R + 1):
        dx = g - _R
        scr_ref[2 * g] = pltpu.roll(qpk, (lanes - dx) % lanes, 2)
        scr_ref[2 * g + 1] = pltpu.roll(qpk, (lanes - (_HW + dx)) % lanes, 2)

    for slot, m, ci in _TAPS:
        qs = scr_ref[slot, :, m:m + _PROWS, :]
        acc = acc + corr_ref[ci].astype(jnp.float32)[None, :, :] * qs

    o_ref[...] = acc * invn_ref[...][None, :, :]


_NC = 7


def kernel(input_, image):
    _, d, h, w = input_.shape
    qlive = input_.reshape(d, _PROWS, 2 * _HW)
    qpk = jnp.pad(qlive, ((0, 0), (_R // 2, _R // 2), (0, 0)))
    nk = _CORR_NP.shape[0]
    out = pl.pallas_call(
        _filter_kernel,
        grid=(d // _NC,),
        in_specs=[
            pl.BlockSpec((_NC, _PROWS, 2 * _HW), lambda i: (i, 0, 0)),
            pl.BlockSpec((_NC, _PPAD, 2 * _HW), lambda i: (i, 0, 0)),
            pl.BlockSpec((2 * _HW, 2 * _HW), lambda i: (0, 0)),
            pl.BlockSpec((_HW, _HW), lambda i: (0, 0)),
            pl.BlockSpec((nk, _PROWS, 2 * _HW), lambda i: (0, 0, 0)),
            pl.BlockSpec((_PROWS, 2 * _HW), lambda i: (0, 0)),
        ],
        out_specs=pl.BlockSpec((_NC, _PROWS, 2 * _HW), lambda i: (i, 0, 0)),
        out_shape=jax.ShapeDtypeStruct((d, _PROWS, 2 * _HW), jnp.float32),
        scratch_shapes=[
            pltpu.VMEM((2 * (2 * _R + 1), _NC, _PPAD, 2 * _HW), jnp.float32)
        ],
    )(qlive, qpk, jnp.asarray(_GB2_NP), jnp.asarray(_GEO_NP),
      jnp.asarray(_CORR_NP), jnp.asarray(_INVN_NP))
    return out.reshape(1, d, h, w)


# speedup vs baseline: 3.5020x; 1.2220x over previous
import numpy as np
import ml_dtypes
import jax
import jax.numpy as jnp
from jax.experimental import pallas as pl
from jax.experimental.pallas import tpu as pltpu

_EPS = float(np.finfo('float').eps)
_HW = 64
_R = 6
_PROWS = _HW // 2
_PPAD = _PROWS + 2 * (_R // 2)


def _build_tables():
    e = (np.arange(_HW, dtype=np.float32) / np.float32(5.0)).astype(np.float32)
    b = e.astype(ml_dtypes.bfloat16).astype(np.float32)
    d2 = (e[:, None] * e[:, None] + e[None, :] * e[None, :]
          - np.float32(2.0) * (b[:, None] * b[None, :])).astype(np.float32)
    gb = np.exp(-0.5 * d2.astype(np.float64)).astype(np.float32)

    offsets = []
    corrs = []
    idx = np.arange(_HW)
    for dy in range(-_R, _R + 1):
        jy = idx + dy
        vy = (jy >= 0) & (jy < _HW)
        u = np.full(_HW, np.inf, np.float32)
        u[vy] = d2[idx[vy], jy[vy]]
        for dx in range(-_R, _R + 1):
            jx = idx + dx
            vx = (jx >= 0) & (jx < _HW)
            v = np.full(_HW, np.inf, np.float32)
            v[vx] = d2[idx[vx], jx[vx]]
            s = u[:, None] + v[None, :]
            neg = s < 0
            if neg.any():
                c = np.zeros((_HW, _HW), np.float64)
                c[neg] = 1.0 - np.exp(-0.5 * s[neg].astype(np.float64))
                offsets.append((dy, dx))
                corrs.append(c.astype(np.float32))

    sg = gb.astype(np.float64).sum(axis=1)
    norm = np.outer(sg, sg) + np.stack(corrs).astype(np.float64).sum(axis=0)
    inv_norm = (1.0 / (norm + _EPS)).astype(np.float32).reshape(_PROWS, 2 * _HW)

    gb2 = np.zeros((2 * _HW, 2 * _HW), np.float32)
    gb2[:_HW, :_HW] = gb
    gb2[_HW:, _HW:] = gb
    geo = np.zeros((_HW, _HW), np.float32)
    geo[:_PROWS, :_PROWS] = gb[0::2, 0::2]
    geo[:_PROWS, _PROWS:] = gb[0::2, 1::2]
    geo[_PROWS:, :_PROWS] = gb[1::2, 1::2]
    geo[_PROWS:, _PROWS:] = gb[1::2, 0::2]

    taps = []
    corr_fields = []
    for (dy, dx), c in zip(offsets, corrs):
        cp = c.reshape(_PROWS, 2 * _HW)
        g = dx + _R
        if dy % 2 == 0:
            taps.append((0, 2 * g, (dy + 2 * (_R // 2)) // 2,
                         len(corr_fields)))
        else:
            taps.append((1, 2 * g + 1, (dy + 2 * (_R // 2) - 1) // 2,
                         len(corr_fields)))
        corr_fields.append(cp)
    corr = np.stack(corr_fields).astype(ml_dtypes.bfloat16)
    return gb2, geo, taps, corr, inv_norm


_GB2_NP, _GEO_NP, _TAPS, _CORR_NP, _INVN_NP = _build_tables()


def _filter_kernel(qlive_ref, gb2_ref, geo_ref, corr_ref, invn_ref,
                   o_ref, scr_ref):
    d = qlive_ref.shape[0]
    lanes = 2 * _HW
    qlive = qlive_ref[...]

    t1p = jnp.dot(qlive.reshape(d * _PROWS, lanes), gb2_ref[...],
                  preferred_element_type=jnp.float32).reshape(d, _PROWS, lanes)
    t1sw = pltpu.roll(t1p, _HW, 2)
    s2 = jnp.concatenate([t1p, t1sw], axis=1)
    lane_is_low = jax.lax.broadcasted_iota(
        jnp.int32, (_PROWS, lanes), 1) < _HW
    geo = geo_ref[...]
    sep = [
        jnp.where(lane_is_low,
                  *jnp.split(jnp.dot(geo, s2[c],
                                     preferred_element_type=jnp.float32), 2))
        for c in range(d)
    ]
    acc = jnp.stack(sep)

    npad = _R // 2
    zpad = jnp.zeros((d, npad, lanes), jnp.float32)
    for g in range(2 * _R + 1):
        dx = g - _R
        for s, sh in ((2 * g, dx), (2 * g + 1, _HW + dx)):
            scr_ref[s, :, :npad, :] = zpad
            scr_ref[s, :, npad:npad + _PROWS, :] = pltpu.roll(
                qlive, (lanes - sh) % lanes, 2)
            scr_ref[s, :, npad + _PROWS:, :] = zpad

    for kind, slot, m, ci in _TAPS:
        qs = scr_ref[slot, :, m:m + _PROWS, :]
        if kind:
            qs = jnp.where(lane_is_low[None, :, :], qs,
                           scr_ref[slot, :, m + 1:m + 1 + _PROWS, :])
        acc = acc + corr_ref[ci].astype(jnp.float32)[None, :, :] * qs

    o_ref[...] = acc * invn_ref[...][None, :, :]


_NC = 7


def kernel(input_, image):
    _, d, h, w = input_.shape
    qlive = input_.reshape(d, _PROWS, 2 * _HW)
    nk = _CORR_NP.shape[0]
    out = pl.pallas_call(
        _filter_kernel,
        grid=(d // _NC,),
        in_specs=[
            pl.BlockSpec((_NC, _PROWS, 2 * _HW), lambda i: (i, 0, 0)),
            pl.BlockSpec((2 * _HW, 2 * _HW), lambda i: (0, 0)),
            pl.BlockSpec((_HW, _HW), lambda i: (0, 0)),
            pl.BlockSpec((nk, _PROWS, 2 * _HW), lambda i: (0, 0, 0)),
            pl.BlockSpec((_PROWS, 2 * _HW), lambda i: (0, 0)),
        ],
        out_specs=pl.BlockSpec((_NC, _PROWS, 2 * _HW), lambda i: (i, 0, 0)),
        out_shape=jax.ShapeDtypeStruct((d, _PROWS, 2 * _HW), jnp.float32),
        scratch_shapes=[
            pltpu.VMEM((2 * (2 * _R + 1), _NC, _PPAD, 2 * _HW), jnp.float32)
        ],
    )(qlive, jnp.asarray(_GB2_NP), jnp.asarray(_GEO_NP),
      jnp.asarray(_CORR_NP), jnp.asarray(_INVN_NP))
    return out.reshape(1, d, h, w)
